# baseline (speedup 1.0000x reference)
import jax
import jax.numpy as jnp
from jax.experimental import pallas as pl
from jax.experimental.pallas import tpu as pltpu

B, S, D, H, E = 1, 2048, 1024, 4096, 8
T = B * S
THRESHOLD = 0.5

HC = 2
HB = H // HC


def _moe_kernel(ew_ref, x_ref, w1_ref, b1_ref, w2_ref, b2_ref, out_ref):
    e = pl.program_id(0)
    hc = pl.program_id(1)

    @pl.when((e == 0) & (hc == 0))
    def _init():
        out_ref[...] = jnp.zeros((T, D), jnp.float32)

    xb = x_ref[...]
    w1 = w1_ref[0].astype(jnp.bfloat16)
    h = jax.lax.dot_general(xb, w1, (((1,), (0,)), ((), ())),
                            preferred_element_type=jnp.float32)
    hb = jnp.maximum(h + b1_ref[0], 0.0).astype(jnp.bfloat16)
    w2 = w2_ref[0].astype(jnp.bfloat16)
    y = jax.lax.dot_general(hb, w2, (((1,), (0,)), ((), ())),
                            preferred_element_type=jnp.float32)
    y = y + jnp.where(hc == 0, 1.0, 0.0) * b2_ref[0]
    out_ref[...] += ew_ref[0] * y


def kernel(x, Wg, bg, W1, b1, W2, b2):
    x_flat = x.reshape(T, D)
    logits = x_flat @ Wg + bg
    probs = jax.nn.sigmoid(logits)
    ew = probs * (probs > THRESHOLD).astype(x_flat.dtype)
    ewt = ew.T.reshape(E, T, 1)
    xb = x_flat.astype(jnp.bfloat16)
    b1r = b1.reshape(E, 1, H)
    b2r = b2.reshape(E, 1, D)
    out = pl.pallas_call(
        _moe_kernel,
        grid=(E, HC),
        in_specs=[
            pl.BlockSpec((1, T, 1), lambda e, hc: (e, 0, 0)),
            pl.BlockSpec((T, D), lambda e, hc: (0, 0)),
            pl.BlockSpec((1, D, HB), lambda e, hc: (e, 0, hc)),
            pl.BlockSpec((1, 1, HB), lambda e, hc: (e, 0, hc)),
            pl.BlockSpec((1, HB, D), lambda e, hc: (e, hc, 0)),
            pl.BlockSpec((1, 1, D), lambda e, hc: (e, 0, 0)),
        ],
        out_specs=pl.BlockSpec((T, D), lambda e, hc: (0, 0)),
        out_shape=jax.ShapeDtypeStruct((T, D), jnp.float32),
        compiler_params=pltpu.CompilerParams(
            dimension_semantics=("arbitrary", "arbitrary")),
    )(ewt, xb, W1, b1r, W2, b2r)
    return out.reshape(B, S, D)


# speedup vs baseline: 1.0108x; 1.0108x over previous
import jax
import jax.numpy as jnp
from jax.experimental import pallas as pl
from jax.experimental.pallas import tpu as pltpu

B, S, D, H, E = 1, 2048, 1024, 4096, 8
T = B * S
THRESHOLD = 0.5

HC = 2
HB = H // HC


def _moe_kernel(ew_ref, x_ref, w1_ref, b1_ref, w2_ref, b2_ref, out_ref):
    e = pl.program_id(0)
    hc = pl.program_id(1)

    @pl.when((e == 0) & (hc == 0))
    def _init():
        out_ref[...] = jnp.zeros((T, D), jnp.float32)

    xb = x_ref[...]
    w1 = w1_ref[0].astype(jnp.bfloat16)
    h = jax.lax.dot_general(xb, w1, (((1,), (0,)), ((), ())),
                            preferred_element_type=jnp.float32)
    hb = jnp.maximum(h + b1_ref[0], 0.0).astype(jnp.bfloat16)
    w2 = w2_ref[0].astype(jnp.bfloat16)
    y = jax.lax.dot_general(hb, w2, (((1,), (0,)), ((), ())),
                            preferred_element_type=jnp.float32)
    y = y + jnp.where(hc == 0, 1.0, 0.0) * b2_ref[0]
    ewb = ew_ref[...]
    lane = jax.lax.broadcasted_iota(jnp.int32, (T, E), 1)
    w = jnp.sum(jnp.where(lane == e, ewb, 0.0), axis=1, keepdims=True)
    out_ref[...] += w * y


def kernel(x, Wg, bg, W1, b1, W2, b2):
    x_flat = x.reshape(T, D)
    logits = x_flat @ Wg + bg
    probs = jax.nn.sigmoid(logits)
    ew = probs * (probs > THRESHOLD).astype(x_flat.dtype)
    xb = x_flat.astype(jnp.bfloat16)
    b1r = b1.reshape(E, 1, H)
    b2r = b2.reshape(E, 1, D)
    out = pl.pallas_call(
        _moe_kernel,
        grid=(E, HC),
        in_specs=[
            pl.BlockSpec((T, E), lambda e, hc: (0, 0)),
            pl.BlockSpec((T, D), lambda e, hc: (0, 0)),
            pl.BlockSpec((1, D, HB), lambda e, hc: (e, 0, hc)),
            pl.BlockSpec((1, 1, HB), lambda e, hc: (e, 0, hc)),
            pl.BlockSpec((1, HB, D), lambda e, hc: (e, hc, 0)),
            pl.BlockSpec((1, 1, D), lambda e, hc: (e, 0, 0)),
        ],
        out_specs=pl.BlockSpec((T, D), lambda e, hc: (0, 0)),
        out_shape=jax.ShapeDtypeStruct((T, D), jnp.float32),
        compiler_params=pltpu.CompilerParams(
            dimension_semantics=("arbitrary", "arbitrary")),
    )(ew, xb, W1, b1r, W2, b2r)
    return out.reshape(B, S, D)
